# baseline (speedup 1.0000x reference)
import functools

import jax
import jax.numpy as jnp
from jax import lax
from jax.experimental import pallas as pl
from jax.experimental.pallas import tpu as pltpu
from jax.experimental.pallas import tpu_sc as plsc

NUM_CORES = 2
NUM_SUBCORES = 16
NUM_WORKERS = NUM_CORES * NUM_SUBCORES

CHUNK = 128
FEATURES = 32


def _build_sc_gather(total_rows: int, features: int, table_rows: int):
    assert total_rows % (NUM_WORKERS * CHUNK) == 0
    rows_per_w = total_rows // NUM_WORKERS
    chunks_per_w = rows_per_w // CHUNK

    mesh = plsc.VectorSubcoreMesh(
        core_axis_name="c", subcore_axis_name="s",
        num_cores=NUM_CORES, num_subcores=NUM_SUBCORES)

    n_chunks = total_rows // CHUNK

    @functools.partial(
        pl.kernel,
        out_type=jax.ShapeDtypeStruct((n_chunks, CHUNK, features), jnp.float32),
        mesh=mesh,
        scratch_types=[
            pltpu.VMEM((chunks_per_w, CHUNK), jnp.int32),
            pltpu.VMEM((CHUNK, features), jnp.float32),
            pltpu.SemaphoreType.DMA,
        ],
        compiler_params=pltpu.CompilerParams(use_tc_tiling_on_sc=False),
    )
    def sc_gather(idx_hbm, tab_hbm, out_hbm, idx_v, buf, gsem):
        wid = lax.axis_index("s") * NUM_CORES + lax.axis_index("c")
        row0 = wid * chunks_per_w
        pltpu.sync_copy(idx_hbm.at[wid], idx_v)

        @pl.loop(0, chunks_per_w)
        def _(j):
            pltpu.async_copy(tab_hbm.at[idx_v.at[j]], buf, gsem).wait()
            pltpu.sync_copy(buf, out_hbm.at[row0 + j])

    return sc_gather


def kernel(inputs, embedding):
    b, s = inputs.shape
    total = b * s
    idx3d = inputs.reshape(
        NUM_WORKERS, total // (NUM_WORKERS * CHUNK), CHUNK).astype(jnp.int32)
    gather = _build_sc_gather(total, embedding.shape[1], embedding.shape[0])
    out = gather(idx3d, embedding)
    return out.reshape(b, s, embedding.shape[1])


# speedup vs baseline: 1.0470x; 1.0470x over previous
import functools

import jax
import jax.numpy as jnp
from jax import lax
from jax.experimental import pallas as pl
from jax.experimental.pallas import tpu as pltpu
from jax.experimental.pallas import tpu_sc as plsc

NUM_CORES = 2
NUM_SUBCORES = 16
NUM_WORKERS = NUM_CORES * NUM_SUBCORES

CHUNK = 128
FEATURES = 32


def _build_sc_gather(total_rows: int, features: int, table_rows: int):
    assert total_rows % (NUM_WORKERS * CHUNK) == 0
    rows_per_w = total_rows // NUM_WORKERS
    chunks_per_w = rows_per_w // CHUNK

    mesh = plsc.VectorSubcoreMesh(
        core_axis_name="c", subcore_axis_name="s",
        num_cores=NUM_CORES, num_subcores=NUM_SUBCORES)

    n_chunks = total_rows // CHUNK
    nbuf = 5
    assert chunks_per_w % nbuf == 0
    outer = chunks_per_w // nbuf

    @functools.partial(
        pl.kernel,
        out_type=jax.ShapeDtypeStruct((n_chunks, CHUNK, features), jnp.float32),
        mesh=mesh,
        scratch_types=[
            pltpu.VMEM((chunks_per_w, CHUNK), jnp.int32),
            pltpu.VMEM((nbuf, CHUNK, features), jnp.float32),
            [pltpu.SemaphoreType.DMA] * nbuf,
            [pltpu.SemaphoreType.DMA] * nbuf,
        ],
        compiler_params=pltpu.CompilerParams(use_tc_tiling_on_sc=False),
    )
    def sc_gather(idx_hbm, tab_hbm, out_hbm, idx_v, buf, gsems, wsems):
        wid = lax.axis_index("s") * NUM_CORES + lax.axis_index("c")
        row0 = wid * chunks_per_w
        pltpu.sync_copy(idx_hbm.at[wid], idx_v)

        for b in range(nbuf):
            pltpu.async_copy(tab_hbm.at[idx_v.at[b]], buf.at[b], gsems[b])

        @pl.loop(0, outer)
        def _(jo):
            j0 = jo * nbuf
            for b in range(nbuf):
                pltpu.make_async_copy(
                    tab_hbm.at[idx_v.at[0]], buf.at[b], gsems[b]).wait()
                pltpu.async_copy(buf.at[b], out_hbm.at[row0 + j0 + b], wsems[b])
            for b in range(nbuf):
                nj = j0 + nbuf + b

                @pl.when(nj < chunks_per_w)
                def _():
                    pltpu.make_async_copy(
                        buf.at[b], out_hbm.at[row0], wsems[b]).wait()
                    pltpu.async_copy(tab_hbm.at[idx_v.at[nj]], buf.at[b],
                                     gsems[b])

        for b in range(nbuf):
            pltpu.make_async_copy(buf.at[b], out_hbm.at[row0], wsems[b]).wait()

    return sc_gather


def kernel(inputs, embedding):
    b, s = inputs.shape
    total = b * s
    idx3d = inputs.reshape(
        NUM_WORKERS, total // (NUM_WORKERS * CHUNK), CHUNK).astype(jnp.int32)
    gather = _build_sc_gather(total, embedding.shape[1], embedding.shape[0])
    out = gather(idx3d, embedding)
    return out.reshape(b, s, embedding.shape[1])
